# baseline (speedup 1.0000x reference)
import functools

import jax
import jax.numpy as jnp
from jax.experimental import pallas as pl
from jax.experimental.pallas import tpu as pltpu

N_EXPERTS = 16
D_MODEL = 1024
D_FF = 4096
FF_TILE = 2048
N_FF_TILES = D_FF // FF_TILE


def _moe_kernel(x_ref, wr_ref, w1_ref, w2_ref, out_ref, combine_ref, acc_ref):
    e = pl.program_id(0)
    f = pl.program_id(1)

    @pl.when(jnp.logical_and(e == 0, f == 0))
    def _router():
        x = x_ref[...]
        logits = jax.lax.dot_general(
            x, wr_ref[...], (((1,), (1,)), ((), ())),
            preferred_element_type=jnp.float32)
        probs = jax.nn.softmax(logits, axis=-1)
        eids = jax.lax.broadcasted_iota(jnp.int32, probs.shape, 1)
        i1 = jnp.argmax(probs, axis=-1)[:, None]
        mask1 = eids == i1
        probs2 = jnp.where(mask1, -jnp.inf, probs)
        i2 = jnp.argmax(probs2, axis=-1)[:, None]
        mask2 = eids == i2
        v1 = jnp.sum(jnp.where(mask1, probs, 0.0), axis=-1, keepdims=True)
        v2 = jnp.sum(jnp.where(mask2, probs, 0.0), axis=-1, keepdims=True)
        norm = v1 + v2
        combine_ref[...] = jnp.where(mask1, v1 / norm,
                                     jnp.where(mask2, v2 / norm, 0.0))
        acc_ref[...] = jnp.zeros_like(acc_ref)

    x = x_ref[...]
    h = jax.lax.dot_general(
        x, w1_ref[0], (((1,), (1,)), ((), ())),
        preferred_element_type=jnp.float32)
    h = h * 0.5 * (1.0 + jax.lax.erf(h * 0.7071067811865476))
    part = jax.lax.dot_general(
        h, w2_ref[0, :, pl.ds(f * FF_TILE, FF_TILE)], (((1,), (1,)), ((), ())),
        preferred_element_type=jnp.float32)
    combine = combine_ref[...]
    eidx = jax.lax.broadcasted_iota(jnp.int32, combine.shape, 1)
    col = jnp.sum(jnp.where(eidx == e, combine, 0.0), axis=1, keepdims=True)
    acc_ref[...] += part * col

    @pl.when(jnp.logical_and(e == N_EXPERTS - 1, f == N_FF_TILES - 1))
    def _done():
        out_ref[...] = acc_ref[...]


@jax.jit
def kernel(x, w_router, w1, w2):
    B, T, C = x.shape
    x_flat = x.reshape(-1, C)
    n = x_flat.shape[0]
    out = pl.pallas_call(
        _moe_kernel,
        grid=(N_EXPERTS, N_FF_TILES),
        in_specs=[
            pl.BlockSpec((n, C), lambda e, f: (0, 0)),
            pl.BlockSpec((N_EXPERTS, C), lambda e, f: (0, 0)),
            pl.BlockSpec((1, FF_TILE, C), lambda e, f: (e, f, 0)),
            pl.BlockSpec((1, C, D_FF), lambda e, f: (e, 0, 0)),
        ],
        out_specs=pl.BlockSpec((n, C), lambda e, f: (0, 0)),
        out_shape=jax.ShapeDtypeStruct((n, C), jnp.float32),
        scratch_shapes=[
            pltpu.VMEM((n, N_EXPERTS), jnp.float32),
            pltpu.VMEM((n, C), jnp.float32),
        ],
    )(x_flat, w_router, w1, w2)
    return out.reshape(B, T, C)


# speedup vs baseline: 1.1979x; 1.1979x over previous
import jax
import jax.numpy as jnp
from jax import lax
from jax.experimental import pallas as pl
from jax.experimental.pallas import tpu as pltpu

N_EXPERTS = 16
D_MODEL = 1024
D_FF = 4096
FF_TILE = 1024
NF = D_FF // FF_TILE


def _moe_kernel(x_ref, wr_ref, w1_hbm, w2_hbm, out_ref,
                w1buf, w2buf, combine_ref, acc_ref, sem1, sem2):
    def w1_copy(e, f, slot):
        return pltpu.make_async_copy(
            w1_hbm.at[e, pl.ds(f * FF_TILE, FF_TILE), :],
            w1buf.at[slot], sem1.at[slot])

    def w2_copy(e, f, slot):
        return pltpu.make_async_copy(
            w2_hbm.at[e, :, pl.ds(f * FF_TILE, FF_TILE)],
            w2buf.at[slot], sem2.at[slot])

    for b in range(NF):
        w1_copy(0, b, b).start()
        w2_copy(0, b, b).start()

    x = x_ref[...]
    logits = lax.dot_general(
        x, wr_ref[...], (((1,), (1,)), ((), ())),
        preferred_element_type=jnp.float32)
    probs = jax.nn.softmax(logits, axis=-1)
    eids = lax.broadcasted_iota(jnp.int32, probs.shape, 1)
    i1 = jnp.argmax(probs, axis=-1)[:, None]
    mask1 = eids == i1
    probs2 = jnp.where(mask1, -jnp.inf, probs)
    i2 = jnp.argmax(probs2, axis=-1)[:, None]
    mask2 = eids == i2
    v1 = jnp.sum(jnp.where(mask1, probs, 0.0), axis=-1, keepdims=True)
    v2 = jnp.sum(jnp.where(mask2, probs, 0.0), axis=-1, keepdims=True)
    norm = v1 + v2
    combine_ref[...] = jnp.where(mask1, v1 / norm,
                                 jnp.where(mask2, v2 / norm, 0.0))
    acc_ref[...] = jnp.zeros_like(acc_ref)

    def per_expert(e, carry):
        combine = combine_ref[...]
        eidx = lax.broadcasted_iota(jnp.int32, combine.shape, 1)
        col = jnp.sum(jnp.where(eidx == e, combine, 0.0),
                      axis=1, keepdims=True)
        for b in range(NF):
            w1_copy(e, b, b).wait()
            w2_copy(e, b, b).wait()
            h = lax.dot_general(
                x, w1buf[b], (((1,), (1,)), ((), ())),
                preferred_element_type=jnp.float32)
            h = h * 0.5 * (1.0 + lax.erf(h * 0.7071067811865476))
            part = lax.dot_general(
                h, w2buf[b], (((1,), (1,)), ((), ())),
                preferred_element_type=jnp.float32)
            acc_ref[...] += part * col

            @pl.when(e + 1 < N_EXPERTS)
            def _refill():
                w1_copy(e + 1, b, b).start()
                w2_copy(e + 1, b, b).start()
        return carry

    lax.fori_loop(0, N_EXPERTS, per_expert, 0)
    out_ref[...] = acc_ref[...]


@jax.jit
def kernel(x, w_router, w1, w2):
    B, T, C = x.shape
    x_flat = x.reshape(-1, C)
    n = x_flat.shape[0]
    out = pl.pallas_call(
        _moe_kernel,
        in_specs=[
            pl.BlockSpec(memory_space=pltpu.VMEM),
            pl.BlockSpec(memory_space=pltpu.VMEM),
            pl.BlockSpec(memory_space=pl.ANY),
            pl.BlockSpec(memory_space=pl.ANY),
        ],
        out_specs=pl.BlockSpec(memory_space=pltpu.VMEM),
        out_shape=jax.ShapeDtypeStruct((n, C), jnp.float32),
        scratch_shapes=[
            pltpu.VMEM((NF, FF_TILE, C), jnp.float32),
            pltpu.VMEM((NF, C, FF_TILE), jnp.float32),
            pltpu.VMEM((n, N_EXPERTS), jnp.float32),
            pltpu.VMEM((n, C), jnp.float32),
            pltpu.SemaphoreType.DMA((NF,)),
            pltpu.SemaphoreType.DMA((NF,)),
        ],
    )(x_flat, w_router, w1, w2)
    return out.reshape(B, T, C)


# speedup vs baseline: 1.1991x; 1.0010x over previous
import jax
import jax.numpy as jnp
from jax import lax
from jax.experimental import pallas as pl
from jax.experimental.pallas import tpu as pltpu

N_EXPERTS = 16
D_MODEL = 1024
D_FF = 4096
FF_TILE = 1024
NF = D_FF // FF_TILE


def _moe_kernel(x_ref, wr_ref, w1_hbm, w2_hbm, out_ref,
                w1buf, w2buf, combine_ref, acc_ref,
                sem1a, sem1b, sem2a, sem2b):
    HF = FF_TILE // 2
    HC = D_MODEL // 2

    def w1_copies(e, f, slot):
        return (
            pltpu.make_async_copy(
                w1_hbm.at[e, pl.ds(f * FF_TILE, HF), :],
                w1buf.at[slot, pl.ds(0, HF)], sem1a.at[slot]),
            pltpu.make_async_copy(
                w1_hbm.at[e, pl.ds(f * FF_TILE + HF, HF), :],
                w1buf.at[slot, pl.ds(HF, HF)], sem1b.at[slot]),
        )

    def w2_copies(e, f, slot):
        return (
            pltpu.make_async_copy(
                w2_hbm.at[e, pl.ds(0, HC), pl.ds(f * FF_TILE, FF_TILE)],
                w2buf.at[slot, pl.ds(0, HC)], sem2a.at[slot]),
            pltpu.make_async_copy(
                w2_hbm.at[e, pl.ds(HC, HC), pl.ds(f * FF_TILE, FF_TILE)],
                w2buf.at[slot, pl.ds(HC, HC)], sem2b.at[slot]),
        )

    def w1_copy(e, f, slot):
        a, b = w1_copies(e, f, slot)

        class _Both:
            def start(self):
                a.start(); b.start()

            def wait(self):
                a.wait(); b.wait()
        return _Both()

    def w2_copy(e, f, slot):
        a, b = w2_copies(e, f, slot)

        class _Both:
            def start(self):
                a.start(); b.start()

            def wait(self):
                a.wait(); b.wait()
        return _Both()

    for b in range(NF):
        w1_copy(0, b, b).start()
        w2_copy(0, b, b).start()

    x = x_ref[...]
    logits = lax.dot_general(
        x, wr_ref[...], (((1,), (1,)), ((), ())),
        preferred_element_type=jnp.float32)
    probs = jax.nn.softmax(logits, axis=-1)
    eids = lax.broadcasted_iota(jnp.int32, probs.shape, 1)
    i1 = jnp.argmax(probs, axis=-1)[:, None]
    mask1 = eids == i1
    probs2 = jnp.where(mask1, -jnp.inf, probs)
    i2 = jnp.argmax(probs2, axis=-1)[:, None]
    mask2 = eids == i2
    v1 = jnp.sum(jnp.where(mask1, probs, 0.0), axis=-1, keepdims=True)
    v2 = jnp.sum(jnp.where(mask2, probs, 0.0), axis=-1, keepdims=True)
    norm = v1 + v2
    combine_ref[...] = jnp.where(mask1, v1 / norm,
                                 jnp.where(mask2, v2 / norm, 0.0))
    acc_ref[...] = jnp.zeros_like(acc_ref)

    def per_expert(e, carry):
        combine = combine_ref[...]
        eidx = lax.broadcasted_iota(jnp.int32, combine.shape, 1)
        col = jnp.sum(jnp.where(eidx == e, combine, 0.0),
                      axis=1, keepdims=True)
        for b in range(NF):
            w1_copy(e, b, b).wait()
            w2_copy(e, b, b).wait()
            h = lax.dot_general(
                x, w1buf[b], (((1,), (1,)), ((), ())),
                preferred_element_type=jnp.float32)
            h = h * 0.5 * (1.0 + lax.erf(h * 0.7071067811865476))
            part = lax.dot_general(
                h, w2buf[b], (((1,), (1,)), ((), ())),
                preferred_element_type=jnp.float32)
            acc_ref[...] += part * col

            @pl.when(e + 1 < N_EXPERTS)
            def _refill():
                w1_copy(e + 1, b, b).start()
                w2_copy(e + 1, b, b).start()
        return carry

    lax.fori_loop(0, N_EXPERTS, per_expert, 0)
    out_ref[...] = acc_ref[...]


@jax.jit
def kernel(x, w_router, w1, w2):
    B, T, C = x.shape
    x_flat = x.reshape(-1, C)
    n = x_flat.shape[0]
    out = pl.pallas_call(
        _moe_kernel,
        in_specs=[
            pl.BlockSpec(memory_space=pltpu.VMEM),
            pl.BlockSpec(memory_space=pltpu.VMEM),
            pl.BlockSpec(memory_space=pl.ANY),
            pl.BlockSpec(memory_space=pl.ANY),
        ],
        out_specs=pl.BlockSpec(memory_space=pltpu.VMEM),
        out_shape=jax.ShapeDtypeStruct((n, C), jnp.float32),
        scratch_shapes=[
            pltpu.VMEM((NF, FF_TILE, C), jnp.float32),
            pltpu.VMEM((NF, C, FF_TILE), jnp.float32),
            pltpu.VMEM((n, N_EXPERTS), jnp.float32),
            pltpu.VMEM((n, C), jnp.float32),
            pltpu.SemaphoreType.DMA((NF,)),
            pltpu.SemaphoreType.DMA((NF,)),
            pltpu.SemaphoreType.DMA((NF,)),
            pltpu.SemaphoreType.DMA((NF,)),
        ],
    )(x_flat, w_router, w1, w2)
    return out.reshape(B, T, C)
